# Initial kernel scaffold; baseline (speedup 1.0000x reference)
import jax
import jax.numpy as jnp
from jax.experimental import pallas as pl


def kernel(x, edge_index, edge_attr, W1, b1, W2, Wih, Whh, bih, bhh, Wig, big):
    raise NotImplementedError("write your pallas kernel here")


# speedup vs baseline: 2.1559x; 2.1559x over previous
import functools

import jax
import jax.numpy as jnp
from jax import lax
from jax.experimental import pallas as pl
from jax.experimental.pallas import tpu as pltpu
from jax.experimental.pallas import tpu_sc as plsc

N = 10000
E = 320000
F = 128
HID = 64
DE = 16
NC = 2
NS = 16
NW = NC * NS
EPT = E // NW
CH = 80
NCHUNK = EPT // CH
NP = 10240
RPT = NP // NS
ZR = 128

_DN = (((1,), (1,)), ((), ()))
_HP = lax.Precision.HIGHEST


def _inorm(t):
    m = jnp.mean(t, axis=1, keepdims=True)
    v = jnp.var(t, axis=1, keepdims=True)
    return (t - m) / jnp.sqrt(v + 1e-05)


def _aggr_body(h_hbm, w_hbm, src_hbm, dst_hbm, out_hbm,
               idx_s, idx_d, hrows, mrows, zbuf, aggr_sh, sem):
    c = lax.axis_index("c")
    s = lax.axis_index("s")

    z16 = jnp.zeros((16,), jnp.float32)

    def _zrow(r, carry):
        for cc in range(F // 16):
            zbuf[r, pl.ds(cc * 16, 16)] = z16
        return carry

    lax.fori_loop(0, ZR, _zrow, 0)
    r0 = s * RPT
    for j in range(RPT // ZR):
        pltpu.sync_copy(zbuf, aggr_sh.at[pl.ds(r0 + j * ZR, ZR), :])
    plsc.subcore_barrier()

    base_e = (c * NS + s) * EPT

    def _chunk(i, carry):
        e0 = base_e + i * CH
        pltpu.sync_copy(src_hbm.at[pl.ds(e0, CH)], idx_s)
        pltpu.sync_copy(dst_hbm.at[pl.ds(e0, CH)], idx_d)
        pltpu.async_copy(h_hbm.at[idx_s], hrows, sem).wait()
        pltpu.sync_copy(w_hbm.at[pl.ds(e0, CH), :], mrows)

        def _mul(r, cr):
            for cc in range(F // 16):
                sl = pl.ds(cc * 16, 16)
                mrows[r, sl] = mrows[r, sl] * hrows[r, sl]
            return cr

        lax.fori_loop(0, CH, _mul, 0)
        pltpu.sync_copy(mrows, aggr_sh.at[idx_d], add=True)
        return carry

    lax.fori_loop(0, NCHUNK, _chunk, 0)
    plsc.subcore_barrier()

    pltpu.sync_copy(aggr_sh.at[pl.ds(r0, RPT), :],
                    out_hbm.at[pl.ds(c * NP + r0, RPT), :])


@functools.cache
def _build_aggr():
    return pl.kernel(
        _aggr_body,
        out_type=jax.ShapeDtypeStruct((NC * NP, F), jnp.float32),
        mesh=plsc.VectorSubcoreMesh(core_axis_name="c", subcore_axis_name="s",
                                    num_cores=NC, num_subcores=NS),
        scratch_types=[
            pltpu.VMEM((CH,), jnp.int32),
            pltpu.VMEM((CH,), jnp.int32),
            pltpu.VMEM((CH, F), jnp.float32),
            pltpu.VMEM((CH, F), jnp.float32),
            pltpu.VMEM((ZR, F), jnp.float32),
            pltpu.VMEM_SHARED((NP, F), jnp.float32),
            pltpu.SemaphoreType.DMA,
        ],
    )


def _deg_body(dst_hbm, out_hbm, idx_d, ones_v, zbuf, deg_sh):
    c = lax.axis_index("c")
    s = lax.axis_index("s")

    o16 = jnp.ones((16,), jnp.float32)
    z16 = jnp.zeros((16,), jnp.float32)

    def _ofill(r, carry):
        ones_v[pl.ds(r * 16, 16)] = o16
        return carry

    lax.fori_loop(0, CH // 16, _ofill, 0)

    def _zfill(r, carry):
        zbuf[pl.ds(r * 16, 16)] = z16
        return carry

    lax.fori_loop(0, RPT // 16, _zfill, 0)
    r0 = s * RPT
    pltpu.sync_copy(zbuf, deg_sh.at[pl.ds(r0, RPT)])
    plsc.subcore_barrier()

    base_e = (c * NS + s) * EPT

    def _chunk(i, carry):
        e0 = base_e + i * CH
        pltpu.sync_copy(dst_hbm.at[pl.ds(e0, CH)], idx_d)
        pltpu.sync_copy(ones_v, deg_sh.at[idx_d], add=True)
        return carry

    lax.fori_loop(0, NCHUNK, _chunk, 0)
    plsc.subcore_barrier()

    pltpu.sync_copy(deg_sh.at[pl.ds(r0, RPT)],
                    out_hbm.at[pl.ds(c * NP + r0, RPT)])


@functools.cache
def _build_deg():
    return pl.kernel(
        _deg_body,
        out_type=jax.ShapeDtypeStruct((NC * NP,), jnp.float32),
        mesh=plsc.VectorSubcoreMesh(core_axis_name="c", subcore_axis_name="s",
                                    num_cores=NC, num_subcores=NS),
        scratch_types=[
            pltpu.VMEM((CH,), jnp.int32),
            pltpu.VMEM((CH,), jnp.float32),
            pltpu.VMEM((RPT,), jnp.float32),
            pltpu.VMEM_SHARED((NP,), jnp.float32),
        ],
    )


BE = 1280


def _w_body(ea_ref, W1_ref, b1_ref, W2_ref, o_ref):
    t = lax.dot_general(ea_ref[...], W1_ref[...], _DN, precision=_HP,
                        preferred_element_type=jnp.float32)
    t = jnp.maximum(t + b1_ref[...], 0.0)
    o_ref[...] = lax.dot_general(t, W2_ref[...], _DN, precision=_HP,
                                 preferred_element_type=jnp.float32)


@functools.cache
def _build_w():
    return pl.pallas_call(
        _w_body,
        grid=(E // BE,),
        in_specs=[
            pl.BlockSpec((BE, DE), lambda i: (i, 0)),
            pl.BlockSpec((HID, DE), lambda i: (0, 0)),
            pl.BlockSpec((1, HID), lambda i: (0, 0)),
            pl.BlockSpec((F, HID), lambda i: (0, 0)),
        ],
        out_specs=pl.BlockSpec((BE, F), lambda i: (i, 0)),
        out_shape=jax.ShapeDtypeStruct((E, F), jnp.float32),
    )


BG = 1000


def _gru_body(h_ref, ap_ref, degp_ref, Wih_ref, Whh_ref, bih_ref, bhh_ref,
              Wig_ref, big_ref, o_ref):
    h = h_ref[...]
    aggr = ap_ref[0] + ap_ref[1]
    deg = degp_ref[0, :, 0:1] + degp_ref[1, :, 0:1]
    aggr = aggr / jnp.maximum(deg, 1.0)

    ig = jax.nn.sigmoid(
        lax.dot_general(h, Wig_ref[...], _DN, precision=_HP,
                        preferred_element_type=jnp.float32) + big_ref[...])
    inp = ig * aggr
    gi = _inorm(lax.dot_general(inp, Wih_ref[...], _DN, precision=_HP,
                                preferred_element_type=jnp.float32))
    gh = _inorm(lax.dot_general(h, Whh_ref[...], _DN, precision=_HP,
                                preferred_element_type=jnp.float32))
    bih = bih_ref[...]
    bhh = bhh_ref[...]
    i_r, i_i, i_n = gi[:, :F], gi[:, F:2 * F], gi[:, 2 * F:]
    h_r, h_i, h_n = gh[:, :F], gh[:, F:2 * F], gh[:, 2 * F:]
    b_r, b_i, b_n = bih[:, :F], bih[:, F:2 * F], bih[:, 2 * F:]
    c_r, c_i, c_n = bhh[:, :F], bhh[:, F:2 * F], bhh[:, 2 * F:]
    resetgate = jax.nn.sigmoid(i_r + b_r + h_r + c_r)
    inputgate = jax.nn.sigmoid(i_i + b_i + h_i + c_i)
    newgate = jnp.tanh(i_n + b_n + resetgate * (h_n + c_n))
    o_ref[...] = newgate + inputgate * (h - newgate)


@functools.cache
def _build_gru():
    return pl.pallas_call(
        _gru_body,
        grid=(N // BG,),
        in_specs=[
            pl.BlockSpec((BG, F), lambda i: (i, 0)),
            pl.BlockSpec((NC, BG, F), lambda i: (0, i, 0)),
            pl.BlockSpec((NC, BG, 1), lambda i: (0, i, 0)),
            pl.BlockSpec((3 * F, F), lambda i: (0, 0)),
            pl.BlockSpec((3 * F, F), lambda i: (0, 0)),
            pl.BlockSpec((1, 3 * F), lambda i: (0, 0)),
            pl.BlockSpec((1, 3 * F), lambda i: (0, 0)),
            pl.BlockSpec((F, F), lambda i: (0, 0)),
            pl.BlockSpec((1, F), lambda i: (0, 0)),
        ],
        out_specs=pl.BlockSpec((BG, F), lambda i: (i, 0)),
        out_shape=jax.ShapeDtypeStruct((N, F), jnp.float32),
    )


def kernel(x, edge_index, edge_attr, W1, b1, W2, Wih, Whh, bih, bhh, Wig, big):
    dst = edge_index[0]
    src = edge_index[1]
    w = _build_w()(edge_attr, W1, b1.reshape(1, HID), W2)
    degp = _build_deg()(dst).reshape(NC, NP, 1)
    bih2 = bih.reshape(1, 3 * F)
    bhh2 = bhh.reshape(1, 3 * F)
    big2 = big.reshape(1, F)
    h = x
    for _ in range(2):
        ap = _build_aggr()(h, w, src, dst).reshape(NC, NP, F)
        h = _build_gru()(h, ap, degp, Wih, Whh, bih2, bhh2, Wig, big2)
    return h


# speedup vs baseline: 2.6396x; 1.2243x over previous
import functools

import jax
import jax.numpy as jnp
from jax import lax
from jax.experimental import pallas as pl
from jax.experimental.pallas import tpu as pltpu
from jax.experimental.pallas import tpu_sc as plsc

N = 10000
E = 320000
F = 128
HID = 64
DE = 16
NC = 2
NS = 16
NW = NC * NS
CH = 32
NCHT = 320
EPW = NCHT * CH
EP = NW * EPW
NP = 10240
RPT = NP // NS
ZR = 128

_DN = (((1,), (1,)), ((), ()))
_HP = lax.Precision.DEFAULT


def _inorm(t):
    m = jnp.mean(t, axis=1, keepdims=True)
    v = jnp.var(t, axis=1, keepdims=True)
    return (t - m) / jnp.sqrt(v + 1e-05)


def _make_aggr_body(with_deg):
    def body(h_hbm, w_hbm, src_hbm, dst_hbm, *rest):
        if with_deg:
            (out_hbm, outd_hbm, idx_s, idx_d, idx_db, h0, h1, w0, w1, m0, m1,
             ones_v, zdeg, deg_sh, aggr_sh, semL0, semL1, semS0, semS1) = rest
        else:
            (out_hbm, idx_s, idx_d, idx_db, h0, h1, w0, w1, m0, m1,
             aggr_sh, semL0, semL1, semS0, semS1) = rest
        hr, wr, mr = (h0, h1), (w0, w1), (m0, m1)
        semL, semS = (semL0, semL1), (semS0, semS1)

        c = lax.axis_index("c")
        s = lax.axis_index("s")
        wid = c * NS + s
        r0 = s * RPT

        pltpu.sync_copy(src_hbm.at[wid], idx_s)
        pltpu.sync_copy(dst_hbm.at[wid], idx_d)

        z16 = jnp.zeros((16,), jnp.float32)

        def _zrow(r, cr):
            for cc in range(F // 16):
                m0[r, pl.ds(cc * 16, 16)] = z16
            return cr

        lax.fori_loop(0, CH, _zrow, 0)

        def _zcopy(j, cr):
            pltpu.sync_copy(m0, aggr_sh.at[pl.ds(r0 + j * CH, CH), :])
            return cr

        lax.fori_loop(0, RPT // CH, _zcopy, 0)

        if with_deg:
            o16 = jnp.ones((16,), jnp.float32)

            def _ofill(r, cr):
                ones_v[pl.ds(r * 16, 16)] = o16
                return cr

            lax.fori_loop(0, CH // 16, _ofill, 0)

            def _zfill(r, cr):
                zdeg[pl.ds(r * 16, 16)] = z16
                return cr

            lax.fori_loop(0, RPT // 16, _zfill, 0)
            pltpu.sync_copy(zdeg, deg_sh.at[pl.ds(r0, RPT)])

        plsc.subcore_barrier()

        def issue_loads(ch, b):
            pltpu.async_copy(h_hbm.at[idx_s.at[pl.ds(ch * CH, CH)]], hr[b],
                             semL[b])
            pltpu.async_copy(w_hbm.at[pl.ds((wid * NCHT + ch) * CH, CH), :],
                             wr[b], semL[b])

        def wait_loads(ch, b):
            pltpu.make_async_copy(h_hbm.at[idx_s.at[pl.ds(ch * CH, CH)]],
                                  hr[b], semL[b]).wait()
            pltpu.make_async_copy(
                w_hbm.at[pl.ds((wid * NCHT + ch) * CH, CH), :], wr[b],
                semL[b]).wait()

        issue_loads(0, 0)
        issue_loads(1, 1)

        def _step(j, carry):
            for b in range(2):
                ch = 2 * j + b
                wait_loads(ch, b)

                @pl.when(j > 0)
                def _wait_prev_scatter(b=b):
                    pltpu.make_async_copy(mr[b], aggr_sh.at[idx_db.at[b]],
                                          semS[b]).wait()

                for g in range(CH // 16):
                    idx_db[b, pl.ds(g * 16, 16)] = (
                        idx_d[pl.ds(ch * CH + g * 16, 16)])

                def _mul(r, cr, b=b):
                    for cc in range(F // 16):
                        sl = pl.ds(cc * 16, 16)
                        mr[b][r, sl] = hr[b][r, sl] * wr[b][r, sl]
                    return cr

                lax.fori_loop(0, CH, _mul, 0)
                pltpu.async_copy(mr[b], aggr_sh.at[idx_db.at[b]], semS[b],
                                 add=True)
                if with_deg:
                    pltpu.sync_copy(ones_v, deg_sh.at[idx_db.at[b]], add=True)

                @pl.when(ch + 2 < NCHT)
                def _issue_next(b=b, ch=ch):
                    issue_loads(ch + 2, b)
            return carry

        lax.fori_loop(0, NCHT // 2, _step, 0)
        for b in range(2):
            pltpu.make_async_copy(mr[b], aggr_sh.at[idx_db.at[b]],
                                  semS[b]).wait()
        plsc.subcore_barrier()

        pltpu.sync_copy(aggr_sh.at[pl.ds(r0, RPT), :],
                        out_hbm.at[pl.ds(c * NP + r0, RPT), :])
        if with_deg:
            pltpu.sync_copy(deg_sh.at[pl.ds(r0, RPT)],
                            outd_hbm.at[pl.ds(c * NP + r0, RPT)])

    return body


@functools.cache
def _build_aggr(with_deg):
    out_type = [jax.ShapeDtypeStruct((NC * NP, F), jnp.float32)]
    scratch = [
        pltpu.VMEM((EPW,), jnp.int32),
        pltpu.VMEM((EPW,), jnp.int32),
        pltpu.VMEM((2, CH), jnp.int32),
        pltpu.VMEM((CH, F), jnp.float32),
        pltpu.VMEM((CH, F), jnp.float32),
        pltpu.VMEM((CH, F), jnp.float32),
        pltpu.VMEM((CH, F), jnp.float32),
        pltpu.VMEM((CH, F), jnp.float32),
        pltpu.VMEM((CH, F), jnp.float32),
    ]
    if with_deg:
        out_type.append(jax.ShapeDtypeStruct((NC * NP,), jnp.float32))
        scratch += [
            pltpu.VMEM((CH,), jnp.float32),
            pltpu.VMEM((RPT,), jnp.float32),
            pltpu.VMEM_SHARED((NP,), jnp.float32),
        ]
    scratch += [
        pltpu.VMEM_SHARED((NP, F), jnp.float32),
        pltpu.SemaphoreType.DMA,
        pltpu.SemaphoreType.DMA,
        pltpu.SemaphoreType.DMA,
        pltpu.SemaphoreType.DMA,
    ]
    return pl.kernel(
        _make_aggr_body(with_deg),
        out_type=tuple(out_type) if with_deg else out_type[0],
        mesh=plsc.VectorSubcoreMesh(core_axis_name="c", subcore_axis_name="s",
                                    num_cores=NC, num_subcores=NS),
        scratch_types=scratch,
    )


BE = 2560


def _w_body(ea_ref, W1_ref, b1_ref, W2_ref, o_ref):
    t = lax.dot_general(ea_ref[...], W1_ref[...], _DN, precision=_HP,
                        preferred_element_type=jnp.float32)
    t = jnp.maximum(t + b1_ref[...], 0.0)
    o_ref[...] = lax.dot_general(t, W2_ref[...], _DN, precision=_HP,
                                 preferred_element_type=jnp.float32)


@functools.cache
def _build_w():
    return pl.pallas_call(
        _w_body,
        grid=(EP // BE,),
        in_specs=[
            pl.BlockSpec((BE, DE), lambda i: (i, 0)),
            pl.BlockSpec((HID, DE), lambda i: (0, 0)),
            pl.BlockSpec((1, HID), lambda i: (0, 0)),
            pl.BlockSpec((F, HID), lambda i: (0, 0)),
        ],
        out_specs=pl.BlockSpec((BE, F), lambda i: (i, 0)),
        out_shape=jax.ShapeDtypeStruct((EP, F), jnp.float32),
    )


BG = 1000


def _gru_body(h_ref, ap_ref, degp_ref, Wih_ref, Whh_ref, bih_ref, bhh_ref,
              Wig_ref, big_ref, o_ref):
    h = h_ref[...]
    aggr = ap_ref[0] + ap_ref[1]
    deg = degp_ref[0, :, 0:1] + degp_ref[1, :, 0:1]
    aggr = aggr / jnp.maximum(deg, 1.0)

    ig = jax.nn.sigmoid(
        lax.dot_general(h, Wig_ref[...], _DN, precision=_HP,
                        preferred_element_type=jnp.float32) + big_ref[...])
    inp = ig * aggr
    gi = _inorm(lax.dot_general(inp, Wih_ref[...], _DN, precision=_HP,
                                preferred_element_type=jnp.float32))
    gh = _inorm(lax.dot_general(h, Whh_ref[...], _DN, precision=_HP,
                                preferred_element_type=jnp.float32))
    bih = bih_ref[...]
    bhh = bhh_ref[...]
    i_r, i_i, i_n = gi[:, :F], gi[:, F:2 * F], gi[:, 2 * F:]
    h_r, h_i, h_n = gh[:, :F], gh[:, F:2 * F], gh[:, 2 * F:]
    b_r, b_i, b_n = bih[:, :F], bih[:, F:2 * F], bih[:, 2 * F:]
    c_r, c_i, c_n = bhh[:, :F], bhh[:, F:2 * F], bhh[:, 2 * F:]
    resetgate = jax.nn.sigmoid(i_r + b_r + h_r + c_r)
    inputgate = jax.nn.sigmoid(i_i + b_i + h_i + c_i)
    newgate = jnp.tanh(i_n + b_n + resetgate * (h_n + c_n))
    o_ref[...] = newgate + inputgate * (h - newgate)


@functools.cache
def _build_gru():
    return pl.pallas_call(
        _gru_body,
        grid=(N // BG,),
        in_specs=[
            pl.BlockSpec((BG, F), lambda i: (i, 0)),
            pl.BlockSpec((NC, BG, F), lambda i: (0, i, 0)),
            pl.BlockSpec((NC, BG, 1), lambda i: (0, i, 0)),
            pl.BlockSpec((3 * F, F), lambda i: (0, 0)),
            pl.BlockSpec((3 * F, F), lambda i: (0, 0)),
            pl.BlockSpec((1, 3 * F), lambda i: (0, 0)),
            pl.BlockSpec((1, 3 * F), lambda i: (0, 0)),
            pl.BlockSpec((F, F), lambda i: (0, 0)),
            pl.BlockSpec((1, F), lambda i: (0, 0)),
        ],
        out_specs=pl.BlockSpec((BG, F), lambda i: (i, 0)),
        out_shape=jax.ShapeDtypeStruct((N, F), jnp.float32),
    )


def kernel(x, edge_index, edge_attr, W1, b1, W2, Wih, Whh, bih, bhh, Wig, big):
    dst = edge_index[0]
    src = edge_index[1]
    pad = EP - E
    dstp = jnp.concatenate(
        [dst, jnp.full((pad,), NP - 1, jnp.int32)]).reshape(NW, EPW)
    srcp = jnp.concatenate(
        [src, jnp.zeros((pad,), jnp.int32)]).reshape(NW, EPW)
    eap = jnp.concatenate(
        [edge_attr, jnp.zeros((pad, DE), jnp.float32)], axis=0)
    w = _build_w()(eap, W1, b1.reshape(1, HID), W2)
    bih2 = bih.reshape(1, 3 * F)
    bhh2 = bhh.reshape(1, 3 * F)
    big2 = big.reshape(1, F)
    ap, degv = _build_aggr(True)(x, w, srcp, dstp)
    degp = degv.reshape(NC, NP, 1)
    h = _build_gru()(x, ap.reshape(NC, NP, F), degp, Wih, Whh, bih2, bhh2,
                     Wig, big2)
    ap2 = _build_aggr(False)(h, w, srcp, dstp)
    h = _build_gru()(h, ap2.reshape(NC, NP, F), degp, Wih, Whh, bih2, bhh2,
                     Wig, big2)
    return h


# speedup vs baseline: 3.1093x; 1.1780x over previous
import functools

import jax
import jax.numpy as jnp
from jax import lax
from jax.experimental import pallas as pl
from jax.experimental.pallas import tpu as pltpu
from jax.experimental.pallas import tpu_sc as plsc

N = 10000
E = 320000
F = 128
HID = 64
DE = 16
NC = 2
NS = 16
NW = NC * NS
CH = 32
NCHT0 = 416
NCHT1 = 224
EPWC = NCHT0 * CH
EP = NS * (NCHT0 + NCHT1) * CH
EP_PAD = EP + EPWC
NP = 10240
RPT = NP // NS
ZR = 128

_DN = (((1,), (1,)), ((), ()))
_HP = lax.Precision.DEFAULT


def _inorm(t):
    m = jnp.mean(t, axis=1, keepdims=True)
    v = jnp.var(t, axis=1, keepdims=True)
    return (t - m) / jnp.sqrt(v + 1e-05)


def _make_aggr_body(with_deg):
    def body(h_hbm, w_hbm, src_hbm, dst_hbm, *rest):
        if with_deg:
            (out_hbm, outd_hbm, idx_s, idx_d, idx_db, h0, h1, w0, w1, m0,
             ones_v, zdeg, deg_sh, aggr_sh, semL0, semL1, semS) = rest
        else:
            (out_hbm, idx_s, idx_d, idx_db, h0, h1, w0, w1, m0,
             aggr_sh, semL0, semL1, semS) = rest
        hr, wr = (h0, h1), (w0, w1)
        semL = (semL0, semL1)

        c = lax.axis_index("c")
        s = lax.axis_index("s")
        r0 = s * RPT
        base_ch = jnp.where(c == 0, s * NCHT0, NS * NCHT0 + s * NCHT1)
        my_ncht = jnp.where(c == 0, NCHT0, NCHT1)

        pltpu.sync_copy(src_hbm.at[pl.ds(base_ch * CH, EPWC)], idx_s)
        pltpu.sync_copy(dst_hbm.at[pl.ds(base_ch * CH, EPWC)], idx_d)

        z16 = jnp.zeros((16,), jnp.float32)

        def _zrow(r, cr):
            for cc in range(F // 16):
                m0[r, pl.ds(cc * 16, 16)] = z16
            return cr

        lax.fori_loop(0, CH, _zrow, 0)

        def _zcopy(j, cr):
            pltpu.sync_copy(m0, aggr_sh.at[pl.ds(r0 + j * CH, CH), :])
            return cr

        lax.fori_loop(0, RPT // CH, _zcopy, 0)

        if with_deg:
            o16 = jnp.ones((16,), jnp.float32)

            def _ofill(r, cr):
                ones_v[pl.ds(r * 16, 16)] = o16
                return cr

            lax.fori_loop(0, CH // 16, _ofill, 0)

            def _zfill(r, cr):
                zdeg[pl.ds(r * 16, 16)] = z16
                return cr

            lax.fori_loop(0, RPT // 16, _zfill, 0)
            pltpu.sync_copy(zdeg, deg_sh.at[pl.ds(r0, RPT)])

        plsc.subcore_barrier()

        def issue_loads(ch, b):
            pltpu.async_copy(h_hbm.at[idx_s.at[pl.ds(ch * CH, CH)]], hr[b],
                             semL[b])
            pltpu.async_copy(w_hbm.at[pl.ds((base_ch + ch) * CH, CH), :],
                             wr[b], semL[b])

        def wait_loads(ch, b):
            pltpu.make_async_copy(h_hbm.at[idx_s.at[pl.ds(ch * CH, CH)]],
                                  hr[b], semL[b]).wait()
            pltpu.make_async_copy(
                w_hbm.at[pl.ds((base_ch + ch) * CH, CH), :], wr[b],
                semL[b]).wait()

        def wait_scatter(b):
            pltpu.make_async_copy(m0, aggr_sh.at[idx_db.at[b]], semS).wait()

        issue_loads(0, 0)
        issue_loads(1, 1)

        def _step(j, carry):
            for b in range(2):
                ch = 2 * j + b
                wait_loads(ch, b)
                if b == 0:
                    @pl.when(j > 0)
                    def _ws():
                        wait_scatter(1)
                else:
                    wait_scatter(0)

                for g in range(CH // 16):
                    idx_db[b, pl.ds(g * 16, 16)] = (
                        idx_d[pl.ds(ch * CH + g * 16, 16)])

                def _mul(r, cr, b=b):
                    for cc in range(F // 16):
                        sl = pl.ds(cc * 16, 16)
                        m0[r, sl] = hr[b][r, sl] * wr[b][r, sl]
                    return cr

                lax.fori_loop(0, CH, _mul, 0)
                pltpu.async_copy(m0, aggr_sh.at[idx_db.at[b]], semS, add=True)
                if with_deg:
                    pltpu.sync_copy(ones_v, deg_sh.at[idx_db.at[b]], add=True)

                @pl.when(ch + 2 < my_ncht)
                def _issue_next(b=b, ch=ch):
                    issue_loads(ch + 2, b)
            return carry

        lax.fori_loop(0, my_ncht // 2, _step, 0)
        wait_scatter(1)
        plsc.subcore_barrier()

        pltpu.sync_copy(aggr_sh.at[pl.ds(r0, RPT), :],
                        out_hbm.at[pl.ds(c * NP + r0, RPT), :])
        if with_deg:
            pltpu.sync_copy(deg_sh.at[pl.ds(r0, RPT)],
                            outd_hbm.at[pl.ds(c * NP + r0, RPT)])

    return body


@functools.cache
def _build_aggr(with_deg):
    out_type = [jax.ShapeDtypeStruct((NC * NP, F), jnp.float32)]
    scratch = [
        pltpu.VMEM((EPWC,), jnp.int32),
        pltpu.VMEM((EPWC,), jnp.int32),
        pltpu.VMEM((2, CH), jnp.int32),
        pltpu.VMEM((CH, F), jnp.float32),
        pltpu.VMEM((CH, F), jnp.float32),
        pltpu.VMEM((CH, F), jnp.float32),
        pltpu.VMEM((CH, F), jnp.float32),
        pltpu.VMEM((CH, F), jnp.float32),
    ]
    if with_deg:
        out_type.append(jax.ShapeDtypeStruct((NC * NP,), jnp.float32))
        scratch += [
            pltpu.VMEM((CH,), jnp.float32),
            pltpu.VMEM((RPT,), jnp.float32),
            pltpu.VMEM_SHARED((NP,), jnp.float32),
        ]
    scratch += [
        pltpu.VMEM_SHARED((NP, F), jnp.float32),
        pltpu.SemaphoreType.DMA,
        pltpu.SemaphoreType.DMA,
        pltpu.SemaphoreType.DMA,
    ]
    return pl.kernel(
        _make_aggr_body(with_deg),
        out_type=tuple(out_type) if with_deg else out_type[0],
        mesh=plsc.VectorSubcoreMesh(core_axis_name="c", subcore_axis_name="s",
                                    num_cores=NC, num_subcores=NS),
        scratch_types=scratch,
    )


BE = 6400


def _w_body(ea_ref, W1_ref, b1_ref, W2_ref, o_ref):
    t = lax.dot_general(ea_ref[...], W1_ref[...], _DN, precision=_HP,
                        preferred_element_type=jnp.float32)
    t = jnp.maximum(t + b1_ref[...], 0.0)
    o_ref[...] = lax.dot_general(t, W2_ref[...], _DN, precision=_HP,
                                 preferred_element_type=jnp.float32)


@functools.cache
def _build_w():
    return pl.pallas_call(
        _w_body,
        grid=(E // BE,),
        in_specs=[
            pl.BlockSpec((BE, DE), lambda i: (i, 0)),
            pl.BlockSpec((HID, DE), lambda i: (0, 0)),
            pl.BlockSpec((1, HID), lambda i: (0, 0)),
            pl.BlockSpec((F, HID), lambda i: (0, 0)),
        ],
        out_specs=pl.BlockSpec((BE, F), lambda i: (i, 0)),
        out_shape=jax.ShapeDtypeStruct((EP, F), jnp.float32),
    )


BG = 1000


def _gru_body(h_ref, ap_ref, degp_ref, Wih_ref, Whh_ref, bih_ref, bhh_ref,
              Wig_ref, big_ref, o_ref):
    h = h_ref[...]
    aggr = ap_ref[0] + ap_ref[1]
    deg = degp_ref[0, :, 0:1] + degp_ref[1, :, 0:1]
    aggr = aggr / jnp.maximum(deg, 1.0)

    ig = jax.nn.sigmoid(
        lax.dot_general(h, Wig_ref[...], _DN, precision=_HP,
                        preferred_element_type=jnp.float32) + big_ref[...])
    inp = ig * aggr
    gi = _inorm(lax.dot_general(inp, Wih_ref[...], _DN, precision=_HP,
                                preferred_element_type=jnp.float32))
    gh = _inorm(lax.dot_general(h, Whh_ref[...], _DN, precision=_HP,
                                preferred_element_type=jnp.float32))
    bih = bih_ref[...]
    bhh = bhh_ref[...]
    i_r, i_i, i_n = gi[:, :F], gi[:, F:2 * F], gi[:, 2 * F:]
    h_r, h_i, h_n = gh[:, :F], gh[:, F:2 * F], gh[:, 2 * F:]
    b_r, b_i, b_n = bih[:, :F], bih[:, F:2 * F], bih[:, 2 * F:]
    c_r, c_i, c_n = bhh[:, :F], bhh[:, F:2 * F], bhh[:, 2 * F:]
    resetgate = jax.nn.sigmoid(i_r + b_r + h_r + c_r)
    inputgate = jax.nn.sigmoid(i_i + b_i + h_i + c_i)
    newgate = jnp.tanh(i_n + b_n + resetgate * (h_n + c_n))
    o_ref[...] = newgate + inputgate * (h - newgate)


@functools.cache
def _build_gru():
    return pl.pallas_call(
        _gru_body,
        grid=(N // BG,),
        in_specs=[
            pl.BlockSpec((BG, F), lambda i: (i, 0)),
            pl.BlockSpec((NC, BG, F), lambda i: (0, i, 0)),
            pl.BlockSpec((NC, BG, 1), lambda i: (0, i, 0)),
            pl.BlockSpec((3 * F, F), lambda i: (0, 0)),
            pl.BlockSpec((3 * F, F), lambda i: (0, 0)),
            pl.BlockSpec((1, 3 * F), lambda i: (0, 0)),
            pl.BlockSpec((1, 3 * F), lambda i: (0, 0)),
            pl.BlockSpec((F, F), lambda i: (0, 0)),
            pl.BlockSpec((1, F), lambda i: (0, 0)),
        ],
        out_specs=pl.BlockSpec((BG, F), lambda i: (i, 0)),
        out_shape=jax.ShapeDtypeStruct((N, F), jnp.float32),
    )


def kernel(x, edge_index, edge_attr, W1, b1, W2, Wih, Whh, bih, bhh, Wig, big):
    dst = edge_index[0]
    src = edge_index[1]
    pad = EP_PAD - E
    dstp = jnp.concatenate([dst, jnp.full((pad,), NP - 1, jnp.int32)])
    srcp = jnp.concatenate([src, jnp.zeros((pad,), jnp.int32)])
    w = _build_w()(edge_attr, W1, b1.reshape(1, HID), W2)
    bih2 = bih.reshape(1, 3 * F)
    bhh2 = bhh.reshape(1, 3 * F)
    big2 = big.reshape(1, F)
    ap, degv = _build_aggr(True)(x, w, srcp, dstp)
    degp = degv.reshape(NC, NP, 1)
    h = _build_gru()(x, ap.reshape(NC, NP, F), degp, Wih, Whh, bih2, bhh2,
                     Wig, big2)
    ap2 = _build_aggr(False)(h, w, srcp, dstp)
    h = _build_gru()(h, ap2.reshape(NC, NP, F), degp, Wih, Whh, bih2, bhh2,
                     Wig, big2)
    return h
